# baseline (speedup 1.0000x reference)
import functools

import jax
import jax.numpy as jnp
from jax import lax
from jax.experimental import pallas as pl
from jax.experimental.pallas import tpu as pltpu
from jax.experimental.pallas import tpu_sc as plsc

F = 26
V = 100000
D = 32
B = 16384
NUM_NUMERIC = 13
H1 = 256
H2 = 64

NGRP = 7
FP = 4 * NGRP
GB = 128
CH = 1024
NG = CH // GB
GR = 4 * B


def _sc_gather_group(cat_ilv, tab_g, nf):
    info = plsc.get_sparse_core_info()
    NC, NS = info.num_cores, info.num_subcores
    NW = NC * NS
    per_w = GR // NW
    nch = per_w // CH

    @functools.partial(
        pl.kernel,
        mesh=plsc.VectorSubcoreMesh(core_axis_name="c", subcore_axis_name="s"),
        out_type=jax.ShapeDtypeStruct((GR, D), jnp.float32),
        scratch_types=[
            pltpu.VMEM((NG, GB), jnp.int32),
            pltpu.VMEM((CH, D), jnp.float32),
            pltpu.SemaphoreType.DMA,
        ],
        compiler_params=pltpu.CompilerParams(use_tc_tiling_on_sc=False),
    )
    def gather_k(cat_hbm, tab_hbm, out_hbm, idx_v, rows_v, sem):
        wid = lax.axis_index("s") * NC + lax.axis_index("c")
        jvec = lax.iota(jnp.int32, 16) % 4
        foff = jnp.where(jvec < nf, jvec * V, 0)

        @pl.loop(0, nch)
        def _chunk(c):
            base = pl.multiple_of(wid * per_w + c * CH, CH)
            pltpu.sync_copy(cat_hbm.at[base // CH], idx_v)

            @pl.loop(0, NG)
            def _row(r):
                @pl.loop(0, GB // 16)
                def _vec(i):
                    sl = (r, pl.ds(i * 16, 16))
                    idx_v[sl] = idx_v[sl] + foff

            copies = [
                pltpu.async_copy(
                    tab_hbm.at[idx_v.at[r]],
                    rows_v.at[pl.ds(r * GB, GB)],
                    sem,
                )
                for r in range(NG)
            ]
            for cp in copies:
                cp.wait()
            pltpu.sync_copy(rows_v, out_hbm.at[pl.ds(base, CH)])

    return gather_k(cat_ilv, tab_g)


def _tc_mlp(xs, num_pad, w1a3, w1b, b1, w2, b2):
    bb = 512
    grid = (B // bb,)

    def body(*refs):
        x_refs = refs[:NGRP]
        n_ref, w1a_ref, w1b_ref, b1_ref, w2_ref, b2_ref, o_ref = refs[NGRP:]
        h = jnp.dot(n_ref[...], w1b_ref[...], preferred_element_type=jnp.float32)
        for g in range(NGRP):
            h += jnp.dot(x_refs[g][...], w1a_ref[g],
                         preferred_element_type=jnp.float32)
        h = jnp.maximum(h + b1_ref[...], 0.0)
        o = jnp.dot(h, w2_ref[...], preferred_element_type=jnp.float32) + b2_ref[...]
        o_ref[...] = jnp.maximum(o, 0.0)

    return pl.pallas_call(
        body,
        grid=grid,
        in_specs=(
            [pl.BlockSpec((bb, 128), lambda i: (i, 0)) for _ in range(NGRP)]
            + [
                pl.BlockSpec((bb, 16), lambda i: (i, 0)),
                pl.BlockSpec((NGRP, 128, H1), lambda i: (0, 0, 0)),
                pl.BlockSpec((16, H1), lambda i: (0, 0)),
                pl.BlockSpec((1, H1), lambda i: (0, 0)),
                pl.BlockSpec((H1, H2), lambda i: (0, 0)),
                pl.BlockSpec((1, H2), lambda i: (0, 0)),
            ]
        ),
        out_specs=pl.BlockSpec((bb, H2), lambda i: (i, 0)),
        out_shape=jax.ShapeDtypeStruct((B, H2), jnp.float32),
        compiler_params=pltpu.CompilerParams(
            dimension_semantics=("arbitrary",),
        ),
    )(*xs, num_pad, w1a3, w1b, b1, w2, b2)


def kernel(cat_input, num_input, tables, W1, b1, W2, b2):
    dummy = (jnp.arange(B, dtype=cat_input.dtype) * 2)[:, None]
    dummy = dummy + jnp.arange(FP - F, dtype=cat_input.dtype)[None, :]
    cat28 = jnp.concatenate([cat_input, dummy % V], axis=1)

    xs = []
    for g in range(NGRP):
        nf = min(4, F - 4 * g)
        tab_g = lax.slice_in_dim(tables, 4 * g, 4 * g + nf, axis=0)
        cat_g = cat28[:, 4 * g : 4 * (g + 1)].reshape(GR // CH, NG, GB)
        emb_g = _sc_gather_group(cat_g, tab_g.reshape(nf * V, D), nf)
        xs.append(emb_g.reshape(B, 4 * D))

    num_pad = jnp.pad(num_input, ((0, 0), (0, 16 - NUM_NUMERIC)))
    w1a3 = jnp.pad(W1[: F * D], ((0, FP * D - F * D), (0, 0)))
    w1a3 = w1a3.reshape(NGRP, 4 * D, H1)
    w1b = jnp.pad(W1[F * D :], ((0, 16 - NUM_NUMERIC), (0, 0)))
    return _tc_mlp(xs, num_pad, w1a3, w1b,
                   b1.reshape(1, H1), W2, b2.reshape(1, H2))


# speedup vs baseline: 1.5318x; 1.5318x over previous
import functools

import jax
import jax.numpy as jnp
from jax import lax
from jax.experimental import pallas as pl
from jax.experimental.pallas import tpu as pltpu
from jax.experimental.pallas import tpu_sc as plsc

F = 26
V = 100000
D = 32
B = 16384
NUM_NUMERIC = 13
H1 = 256
H2 = 64

NGRP = 7
FP = 4 * NGRP
GB = 128
CH = 2048
NG = CH // GB
TOTR = NGRP * B * 4


def _sc_gather(cat_ilv, tables_flat):
    info = plsc.get_sparse_core_info()
    NC, NS = info.num_cores, info.num_subcores
    NW = NC * NS
    per_w = TOTR // NW
    nch = per_w // CH
    rows_per_grp = B * 4

    @functools.partial(
        pl.kernel,
        mesh=plsc.VectorSubcoreMesh(core_axis_name="c", subcore_axis_name="s"),
        out_type=jax.ShapeDtypeStruct((TOTR, D), jnp.float32),
        scratch_types=[
            pltpu.VMEM((NG, GB), jnp.int32),
            pltpu.VMEM((CH, D), jnp.float32),
            pltpu.SemaphoreType.DMA,
        ],
        compiler_params=pltpu.CompilerParams(use_tc_tiling_on_sc=False),
    )
    def gather_k(cat_hbm, tab_hbm, out_hbm, idx_v, rows_v, sem):
        wid = lax.axis_index("s") * NC + lax.axis_index("c")
        jvec = lax.iota(jnp.int32, 16) % 4

        @pl.loop(0, nch)
        def _chunk(c):
            base = pl.multiple_of(wid * per_w + c * CH, CH)
            pltpu.sync_copy(cat_hbm.at[base // CH], idx_v)

            fvec = (base // rows_per_grp) * 4 + jvec
            foff = jnp.where(fvec < F, fvec * V, 0)

            @pl.loop(0, NG)
            def _row(r):
                @pl.loop(0, GB // 16)
                def _vec(i):
                    sl = (r, pl.ds(i * 16, 16))
                    idx_v[sl] = idx_v[sl] + foff

            copies = [
                pltpu.async_copy(
                    tab_hbm.at[idx_v.at[r]],
                    rows_v.at[pl.ds(r * GB, GB)],
                    sem,
                )
                for r in range(NG)
            ]
            for cp in copies:
                cp.wait()
            pltpu.sync_copy(rows_v, out_hbm.at[pl.ds(base, CH)])

    return gather_k(cat_ilv, tables_flat)


def _tc_mlp(x3, num_pad, w1a3, w1b, b1, w2, b2):
    bb = 512
    grid = (B // bb,)

    def body(x_ref, n_ref, w1a_ref, w1b_ref, b1_ref, w2_ref, b2_ref, o_ref):
        h = jnp.dot(n_ref[...], w1b_ref[...], preferred_element_type=jnp.float32)
        for g in range(NGRP):
            h += jnp.dot(x_ref[g], w1a_ref[g],
                         preferred_element_type=jnp.float32)
        h = jnp.maximum(h + b1_ref[...], 0.0)
        o = jnp.dot(h, w2_ref[...], preferred_element_type=jnp.float32) + b2_ref[...]
        o_ref[...] = jnp.maximum(o, 0.0)

    return pl.pallas_call(
        body,
        grid=grid,
        in_specs=[
            pl.BlockSpec((NGRP, bb, 128), lambda i: (0, i, 0)),
            pl.BlockSpec((bb, 16), lambda i: (i, 0)),
            pl.BlockSpec((NGRP, 128, H1), lambda i: (0, 0, 0)),
            pl.BlockSpec((16, H1), lambda i: (0, 0)),
            pl.BlockSpec((1, H1), lambda i: (0, 0)),
            pl.BlockSpec((H1, H2), lambda i: (0, 0)),
            pl.BlockSpec((1, H2), lambda i: (0, 0)),
        ],
        out_specs=pl.BlockSpec((bb, H2), lambda i: (i, 0)),
        out_shape=jax.ShapeDtypeStruct((B, H2), jnp.float32),
        compiler_params=pltpu.CompilerParams(
            dimension_semantics=("arbitrary",),
        ),
    )(x3, num_pad, w1a3, w1b, b1, w2, b2)


def kernel(cat_input, num_input, tables, W1, b1, W2, b2):
    dummy = (jnp.arange(B, dtype=cat_input.dtype) * 2)[:, None]
    dummy = dummy + jnp.arange(FP - F, dtype=cat_input.dtype)[None, :]
    cat_ilv = jnp.concatenate([cat_input, dummy % V], axis=1)
    cat_ilv = cat_ilv.reshape(B, NGRP, 4).transpose(1, 0, 2)
    cat_ilv = cat_ilv.reshape(TOTR // CH, NG, GB)
    tables_flat = tables.reshape(F * V, D)

    embeds = _sc_gather(cat_ilv, tables_flat)
    x3 = embeds.reshape(NGRP, B, 4 * D)

    num_pad = jnp.pad(num_input, ((0, 0), (0, 16 - NUM_NUMERIC)))
    w1a3 = jnp.pad(W1[: F * D], ((0, FP * D - F * D), (0, 0)))
    w1a3 = w1a3.reshape(NGRP, 4 * D, H1)
    w1b = jnp.pad(W1[F * D :], ((0, 16 - NUM_NUMERIC), (0, 0)))
    return _tc_mlp(x3, num_pad, w1a3, w1b,
                   b1.reshape(1, H1), W2, b2.reshape(1, H2))
